# baseline (device time: 234289 ns/iter reference)
import jax
import jax.numpy as jnp
from jax import lax
from jax.experimental import pallas as pl
from jax.experimental.pallas import tpu as pltpu

N_DEV = 32
ROWS = 1024
CH = ROWS // N_DEV
DH = 128
SCALE = 0.08838834764831843


def _allreduce_body(p_ref, out_ref, rbuf, ss_rs, rs_rs, ss_ag, rs_ag):
    my = lax.axis_index("i")
    left = lax.rem(my - 1 + N_DEV, N_DEV)
    right = lax.rem(my + 1, N_DEV)

    barrier = pltpu.get_barrier_semaphore()
    for nbr in (left, right):
        pl.semaphore_signal(
            barrier, inc=1, device_id=(nbr,), device_id_type=pl.DeviceIdType.MESH
        )
    pl.semaphore_wait(barrier, 2)

    out_ref[...] = p_ref[...]

    for s in range(N_DEV - 1):
        c_send = lax.rem(my - s + 2 * N_DEV, N_DEV)
        c_recv = lax.rem(my - s - 1 + 2 * N_DEV, N_DEV)
        rdma = pltpu.make_async_remote_copy(
            src_ref=out_ref.at[pl.ds(c_send * CH, CH), :],
            dst_ref=rbuf.at[s],
            send_sem=ss_rs.at[s],
            recv_sem=rs_rs.at[s],
            device_id=(right,),
            device_id_type=pl.DeviceIdType.MESH,
        )
        rdma.start()
        rdma.wait()
        out_ref[pl.ds(c_recv * CH, CH), :] = (
            out_ref[pl.ds(c_recv * CH, CH), :] + rbuf[s]
        )

    for s in range(N_DEV - 1):
        c_send = lax.rem(my + 1 - s + 2 * N_DEV, N_DEV)
        rdma = pltpu.make_async_remote_copy(
            src_ref=out_ref.at[pl.ds(c_send * CH, CH), :],
            dst_ref=out_ref.at[pl.ds(c_send * CH, CH), :],
            send_sem=ss_ag.at[s],
            recv_sem=rs_ag.at[s],
            device_id=(right,),
            device_id_type=pl.DeviceIdType.MESH,
        )
        rdma.start()
        rdma.wait()


def _ring_allreduce(partial):
    return pl.pallas_call(
        _allreduce_body,
        out_shape=jax.ShapeDtypeStruct((ROWS, ROWS), jnp.float32),
        in_specs=[pl.BlockSpec(memory_space=pltpu.VMEM)],
        out_specs=pl.BlockSpec(memory_space=pltpu.VMEM),
        scratch_shapes=[
            pltpu.VMEM((N_DEV - 1, CH, ROWS), jnp.float32),
            pltpu.SemaphoreType.DMA((N_DEV - 1,)),
            pltpu.SemaphoreType.DMA((N_DEV - 1,)),
            pltpu.SemaphoreType.DMA((N_DEV - 1,)),
            pltpu.SemaphoreType.DMA((N_DEV - 1,)),
        ],
        compiler_params=pltpu.CompilerParams(collective_id=0),
    )(partial)


def kernel(x, Wq, K_ext, V_ext, Wo):
    my = lax.axis_index("i")
    bf = jnp.bfloat16
    hl = Wq.shape[1] // DH

    x2 = x[0].astype(bf)
    Q = jnp.dot(x2, Wq.astype(bf), preferred_element_type=jnp.float32)
    Q = Q.reshape(ROWS, hl, DH).astype(bf)
    K = lax.dynamic_slice_in_dim(K_ext[0], my * hl, hl, axis=1).astype(bf)
    V = lax.dynamic_slice_in_dim(V_ext[0], my * hl, hl, axis=1).astype(bf)

    def group(t):
        t = t.reshape(4, 4, 64, hl, DH)
        return t.transpose(1, 0, 2, 3, 4).reshape(4, 256, hl, DH)

    Qg, Kg, Vg = group(Q), group(K), group(V)
    scores = (
        jnp.einsum("gihd,gjhd->ghij", Qg, Kg, preferred_element_type=jnp.float32)
        * SCALE
    )
    w = jax.nn.softmax(scores, axis=-1).astype(bf)
    ctx = jnp.einsum("ghij,gjhd->gihd", w, Vg, preferred_element_type=jnp.float32)
    ctx = (
        ctx.reshape(4, 4, 64, hl, DH)
        .transpose(1, 0, 2, 3, 4)
        .reshape(ROWS, hl * DH)
        .astype(bf)
    )
    partial = jnp.dot(ctx, Wo.astype(bf), preferred_element_type=jnp.float32)

    out = _ring_allreduce(partial)
    return out.reshape(1, ROWS, ROWS)


# device time: 150045 ns/iter; 1.5615x vs baseline; 1.5615x over previous
import jax
import jax.numpy as jnp
from jax import lax
from jax.experimental import pallas as pl
from jax.experimental.pallas import tpu as pltpu

N_DEV = 32
P = 8
Z = 4
ROWS = 1024
HALF = 512
PCH = 64
ZCH = 16
NSTEP = (P - 1) + (Z - 1) + (Z - 1) + (P - 1)
DH = 128
SCALE = 0.08838834764831843


def _ar_body(p_ref, out_ref, rb_cw, rb_ccw, zb_cw, zb_ccw, ss_cw, rs_cw, ss_ccw, rs_ccw):
    my = lax.axis_index("i")
    z = my // P
    q = lax.rem(my, P)
    pnext = z * P + lax.rem(q + 1, P)
    pprev = z * P + lax.rem(q - 1 + P, P)
    znext = lax.rem(z + 1, Z) * P + q
    zprev = lax.rem(z - 1 + Z, Z) * P + q

    barrier = pltpu.get_barrier_semaphore()
    for nbr in (pnext, pprev, znext, zprev):
        pl.semaphore_signal(
            barrier, inc=1, device_id=(nbr,), device_id_type=pl.DeviceIdType.MESH
        )
    pl.semaphore_wait(barrier, 4)

    out_ref[...] = p_ref[...]

    pending = []

    def start(src, dst, ssem, rsem, target):
        r = pltpu.make_async_remote_copy(
            src_ref=src,
            dst_ref=dst,
            send_sem=ssem,
            recv_sem=rsem,
            device_id=(target,),
            device_id_type=pl.DeviceIdType.MESH,
        )
        r.start()
        pending.append(r)
        return r

    k = 0

    for s in range(P - 1):
        c_cw = lax.rem(q - s + 2 * P, P)
        c_ccw = lax.rem(q + s, P)
        d1 = start(
            out_ref.at[pl.ds(c_cw * PCH, PCH), :], rb_cw.at[s],
            ss_cw.at[k], rs_cw.at[k], pnext,
        )
        d2 = start(
            out_ref.at[pl.ds(HALF + c_ccw * PCH, PCH), :], rb_ccw.at[s],
            ss_ccw.at[k], rs_ccw.at[k], pprev,
        )
        d1.wait_recv()
        d2.wait_recv()
        r_cw = lax.rem(q - s - 1 + 2 * P, P)
        r_ccw = lax.rem(q + s + 1, P)
        out_ref[pl.ds(r_cw * PCH, PCH), :] = (
            out_ref[pl.ds(r_cw * PCH, PCH), :] + rb_cw[s]
        )
        out_ref[pl.ds(HALF + r_ccw * PCH, PCH), :] = (
            out_ref[pl.ds(HALF + r_ccw * PCH, PCH), :] + rb_ccw[s]
        )
        k += 1

    b_cw = lax.rem(q + 1, P) * PCH
    b_ccw = HALF + lax.rem(q - 1 + P, P) * PCH

    for t in range(Z - 1):
        j_cw = lax.rem(z - t + 2 * Z, Z)
        j_ccw = lax.rem(z + t, Z)
        d1 = start(
            out_ref.at[pl.ds(b_cw + j_cw * ZCH, ZCH), :], zb_cw.at[t],
            ss_cw.at[k], rs_cw.at[k], znext,
        )
        d2 = start(
            out_ref.at[pl.ds(b_ccw + j_ccw * ZCH, ZCH), :], zb_ccw.at[t],
            ss_ccw.at[k], rs_ccw.at[k], zprev,
        )
        d1.wait_recv()
        d2.wait_recv()
        jr_cw = lax.rem(z - t - 1 + 2 * Z, Z)
        jr_ccw = lax.rem(z + t + 1, Z)
        out_ref[pl.ds(b_cw + jr_cw * ZCH, ZCH), :] = (
            out_ref[pl.ds(b_cw + jr_cw * ZCH, ZCH), :] + zb_cw[t]
        )
        out_ref[pl.ds(b_ccw + jr_ccw * ZCH, ZCH), :] = (
            out_ref[pl.ds(b_ccw + jr_ccw * ZCH, ZCH), :] + zb_ccw[t]
        )
        k += 1

    for t in range(Z - 1):
        j_cw = lax.rem(z + 1 - t + 2 * Z, Z)
        j_ccw = lax.rem(z - 1 + t + 2 * Z, Z)
        rows_cw = b_cw + j_cw * ZCH
        rows_ccw = b_ccw + j_ccw * ZCH
        d1 = start(
            out_ref.at[pl.ds(rows_cw, ZCH), :], out_ref.at[pl.ds(rows_cw, ZCH), :],
            ss_cw.at[k], rs_cw.at[k], znext,
        )
        d2 = start(
            out_ref.at[pl.ds(rows_ccw, ZCH), :], out_ref.at[pl.ds(rows_ccw, ZCH), :],
            ss_ccw.at[k], rs_ccw.at[k], zprev,
        )
        d1.wait_recv()
        d2.wait_recv()
        k += 1

    for s in range(P - 1):
        c_cw = lax.rem(q + 1 - s + 2 * P, P)
        c_ccw = lax.rem(q - 1 + s + P, P)
        rows_cw = c_cw * PCH
        rows_ccw = HALF + c_ccw * PCH
        d1 = start(
            out_ref.at[pl.ds(rows_cw, PCH), :], out_ref.at[pl.ds(rows_cw, PCH), :],
            ss_cw.at[k], rs_cw.at[k], pnext,
        )
        d2 = start(
            out_ref.at[pl.ds(rows_ccw, PCH), :], out_ref.at[pl.ds(rows_ccw, PCH), :],
            ss_ccw.at[k], rs_ccw.at[k], pprev,
        )
        d1.wait_recv()
        d2.wait_recv()
        k += 1

    for r in pending:
        r.wait_send()


def _hier_allreduce(partial):
    return pl.pallas_call(
        _ar_body,
        out_shape=jax.ShapeDtypeStruct((ROWS, ROWS), jnp.float32),
        in_specs=[pl.BlockSpec(memory_space=pltpu.VMEM)],
        out_specs=pl.BlockSpec(memory_space=pltpu.VMEM),
        scratch_shapes=[
            pltpu.VMEM((P - 1, PCH, ROWS), jnp.float32),
            pltpu.VMEM((P - 1, PCH, ROWS), jnp.float32),
            pltpu.VMEM((Z - 1, ZCH, ROWS), jnp.float32),
            pltpu.VMEM((Z - 1, ZCH, ROWS), jnp.float32),
            pltpu.SemaphoreType.DMA((NSTEP,)),
            pltpu.SemaphoreType.DMA((NSTEP,)),
            pltpu.SemaphoreType.DMA((NSTEP,)),
            pltpu.SemaphoreType.DMA((NSTEP,)),
        ],
        compiler_params=pltpu.CompilerParams(collective_id=0),
    )(partial)


def kernel(x, Wq, K_ext, V_ext, Wo):
    my = lax.axis_index("i")
    bf = jnp.bfloat16
    hl = Wq.shape[1] // DH

    x2 = x[0].astype(bf)
    Q = jnp.dot(x2, Wq.astype(bf), preferred_element_type=jnp.float32)
    Q = Q.reshape(ROWS, hl, DH).astype(bf)
    K = lax.dynamic_slice_in_dim(K_ext[0], my * hl, hl, axis=1).astype(bf)
    V = lax.dynamic_slice_in_dim(V_ext[0], my * hl, hl, axis=1).astype(bf)

    def group(t):
        t = t.reshape(4, 4, 64, hl, DH)
        return t.transpose(1, 0, 2, 3, 4).reshape(4, 256, hl, DH)

    Qg, Kg, Vg = group(Q), group(K), group(V)
    scores = (
        jnp.einsum("gihd,gjhd->ghij", Qg, Kg, preferred_element_type=jnp.float32)
        * SCALE
    )
    w = jax.nn.softmax(scores, axis=-1).astype(bf)
    ctx = jnp.einsum("ghij,gjhd->gihd", w, Vg, preferred_element_type=jnp.float32)
    ctx = (
        ctx.reshape(4, 4, 64, hl, DH)
        .transpose(1, 0, 2, 3, 4)
        .reshape(ROWS, hl * DH)
        .astype(bf)
    )
    partial = jnp.dot(ctx, Wo.astype(bf), preferred_element_type=jnp.float32)

    out = _hier_allreduce(partial)
    return out.reshape(1, ROWS, ROWS)


# device time: 110565 ns/iter; 2.1190x vs baseline; 1.3571x over previous
import jax
import jax.numpy as jnp
from jax import lax
from jax.experimental import pallas as pl
from jax.experimental.pallas import tpu as pltpu

N_DEV = 32
P = 8
Z = 4
ROWS = 1024
HALF = 512
PCH = 64
ZCH = 16
NSTEP = (P - 1) + (Z - 1) + (Z - 1) + (P - 1)
DH = 128
SCALE = 0.08838834764831843


def _ar_body(
    p_ref, out_ref,
    sb_cw, rb_cw, sb_ccw, rb_ccw,
    zsb_cw, zb_cw, zsb_ccw, zb_ccw,
    zgs_cw, zgb_cw, zgs_ccw, zgb_ccw,
    pgs_cw, pgb_cw, pgs_ccw, pgb_ccw,
    ss_cw, rs_cw, ss_ccw, rs_ccw,
):
    my = lax.axis_index("i")
    z = my // P
    q = lax.rem(my, P)
    pnext = z * P + lax.rem(q + 1, P)
    pprev = z * P + lax.rem(q - 1 + P, P)
    znext = lax.rem(z + 1, Z) * P + q
    zprev = lax.rem(z - 1 + Z, Z) * P + q

    barrier = pltpu.get_barrier_semaphore()
    for nbr in (pnext, pprev, znext, zprev):
        pl.semaphore_signal(
            barrier, inc=1, device_id=(nbr,), device_id_type=pl.DeviceIdType.MESH
        )
    pl.semaphore_wait(barrier, 4)

    out_ref[...] = p_ref[...]

    pending = []

    def start(src, dst, ssem, rsem, target):
        r = pltpu.make_async_remote_copy(
            src_ref=src,
            dst_ref=dst,
            send_sem=ssem,
            recv_sem=rsem,
            device_id=(target,),
            device_id_type=pl.DeviceIdType.MESH,
        )
        r.start()
        pending.append(r)
        return r

    k = 0

    for s in range(P - 1):
        c_cw = lax.rem(q - s + 2 * P, P)
        c_ccw = lax.rem(q + s, P)
        sb_cw[s, :, :] = out_ref[pl.ds(c_cw * PCH, PCH), :].astype(jnp.bfloat16)
        sb_ccw[s, :, :] = out_ref[pl.ds(HALF + c_ccw * PCH, PCH), :].astype(
            jnp.bfloat16
        )
        d1 = start(sb_cw.at[s], rb_cw.at[s], ss_cw.at[k], rs_cw.at[k], pnext)
        d2 = start(sb_ccw.at[s], rb_ccw.at[s], ss_ccw.at[k], rs_ccw.at[k], pprev)
        d1.wait_recv()
        d2.wait_recv()
        r_cw = lax.rem(q - s - 1 + 2 * P, P)
        r_ccw = lax.rem(q + s + 1, P)
        out_ref[pl.ds(r_cw * PCH, PCH), :] = (
            out_ref[pl.ds(r_cw * PCH, PCH), :] + rb_cw[s].astype(jnp.float32)
        )
        out_ref[pl.ds(HALF + r_ccw * PCH, PCH), :] = (
            out_ref[pl.ds(HALF + r_ccw * PCH, PCH), :]
            + rb_ccw[s].astype(jnp.float32)
        )
        k += 1

    b_cw = lax.rem(q + 1, P) * PCH
    b_ccw = HALF + lax.rem(q - 1 + P, P) * PCH

    for t in range(Z - 1):
        j_cw = lax.rem(z - t + 2 * Z, Z)
        j_ccw = lax.rem(z + t, Z)
        zsb_cw[t, :, :] = out_ref[pl.ds(b_cw + j_cw * ZCH, ZCH), :].astype(
            jnp.bfloat16
        )
        zsb_ccw[t, :, :] = out_ref[pl.ds(b_ccw + j_ccw * ZCH, ZCH), :].astype(
            jnp.bfloat16
        )
        d1 = start(zsb_cw.at[t], zb_cw.at[t], ss_cw.at[k], rs_cw.at[k], znext)
        d2 = start(zsb_ccw.at[t], zb_ccw.at[t], ss_ccw.at[k], rs_ccw.at[k], zprev)
        d1.wait_recv()
        d2.wait_recv()
        jr_cw = lax.rem(z - t - 1 + 2 * Z, Z)
        jr_ccw = lax.rem(z + t + 1, Z)
        out_ref[pl.ds(b_cw + jr_cw * ZCH, ZCH), :] = (
            out_ref[pl.ds(b_cw + jr_cw * ZCH, ZCH), :] + zb_cw[t].astype(jnp.float32)
        )
        out_ref[pl.ds(b_ccw + jr_ccw * ZCH, ZCH), :] = (
            out_ref[pl.ds(b_ccw + jr_ccw * ZCH, ZCH), :]
            + zb_ccw[t].astype(jnp.float32)
        )
        k += 1

    zgs_cw[...] = out_ref[pl.ds(b_cw + lax.rem(z + 1, Z) * ZCH, ZCH), :].astype(
        jnp.bfloat16
    )
    zgs_ccw[...] = out_ref[
        pl.ds(b_ccw + lax.rem(z - 1 + Z, Z) * ZCH, ZCH), :
    ].astype(jnp.bfloat16)
    for t in range(Z - 1):
        src_cw = zgs_cw if t == 0 else zgb_cw.at[t - 1]
        src_ccw = zgs_ccw if t == 0 else zgb_ccw.at[t - 1]
        d1 = start(src_cw, zgb_cw.at[t], ss_cw.at[k], rs_cw.at[k], znext)
        d2 = start(src_ccw, zgb_ccw.at[t], ss_ccw.at[k], rs_ccw.at[k], zprev)
        d1.wait_recv()
        d2.wait_recv()
        jr_cw = lax.rem(z - t + 2 * Z, Z)
        jr_ccw = lax.rem(z + t + Z, Z)
        out_ref[pl.ds(b_cw + jr_cw * ZCH, ZCH), :] = zgb_cw[t].astype(jnp.float32)
        out_ref[pl.ds(b_ccw + jr_ccw * ZCH, ZCH), :] = zgb_ccw[t].astype(
            jnp.float32
        )
        k += 1

    pgs_cw[...] = out_ref[pl.ds(b_cw, PCH), :].astype(jnp.bfloat16)
    pgs_ccw[...] = out_ref[pl.ds(b_ccw, PCH), :].astype(jnp.bfloat16)
    for s in range(P - 1):
        src_cw = pgs_cw if s == 0 else pgb_cw.at[s - 1]
        src_ccw = pgs_ccw if s == 0 else pgb_ccw.at[s - 1]
        d1 = start(src_cw, pgb_cw.at[s], ss_cw.at[k], rs_cw.at[k], pnext)
        d2 = start(src_ccw, pgb_ccw.at[s], ss_ccw.at[k], rs_ccw.at[k], pprev)
        d1.wait_recv()
        d2.wait_recv()
        r_cw = lax.rem(q - s + 2 * P, P)
        r_ccw = lax.rem(q + s + P, P)
        out_ref[pl.ds(r_cw * PCH, PCH), :] = pgb_cw[s].astype(jnp.float32)
        out_ref[pl.ds(HALF + r_ccw * PCH, PCH), :] = pgb_ccw[s].astype(jnp.float32)
        k += 1

    for r in pending:
        r.wait_send()


def _hier_allreduce(partial):
    return pl.pallas_call(
        _ar_body,
        out_shape=jax.ShapeDtypeStruct((ROWS, ROWS), jnp.float32),
        in_specs=[pl.BlockSpec(memory_space=pltpu.VMEM)],
        out_specs=pl.BlockSpec(memory_space=pltpu.VMEM),
        scratch_shapes=[
            pltpu.VMEM((P - 1, PCH, ROWS), jnp.bfloat16),
            pltpu.VMEM((P - 1, PCH, ROWS), jnp.bfloat16),
            pltpu.VMEM((P - 1, PCH, ROWS), jnp.bfloat16),
            pltpu.VMEM((P - 1, PCH, ROWS), jnp.bfloat16),
            pltpu.VMEM((Z - 1, ZCH, ROWS), jnp.bfloat16),
            pltpu.VMEM((Z - 1, ZCH, ROWS), jnp.bfloat16),
            pltpu.VMEM((Z - 1, ZCH, ROWS), jnp.bfloat16),
            pltpu.VMEM((Z - 1, ZCH, ROWS), jnp.bfloat16),
            pltpu.VMEM((ZCH, ROWS), jnp.bfloat16),
            pltpu.VMEM((Z - 1, ZCH, ROWS), jnp.bfloat16),
            pltpu.VMEM((ZCH, ROWS), jnp.bfloat16),
            pltpu.VMEM((Z - 1, ZCH, ROWS), jnp.bfloat16),
            pltpu.VMEM((PCH, ROWS), jnp.bfloat16),
            pltpu.VMEM((P - 1, PCH, ROWS), jnp.bfloat16),
            pltpu.VMEM((PCH, ROWS), jnp.bfloat16),
            pltpu.VMEM((P - 1, PCH, ROWS), jnp.bfloat16),
            pltpu.SemaphoreType.DMA((NSTEP,)),
            pltpu.SemaphoreType.DMA((NSTEP,)),
            pltpu.SemaphoreType.DMA((NSTEP,)),
            pltpu.SemaphoreType.DMA((NSTEP,)),
        ],
        compiler_params=pltpu.CompilerParams(collective_id=0),
    )(partial)


def kernel(x, Wq, K_ext, V_ext, Wo):
    my = lax.axis_index("i")
    bf = jnp.bfloat16
    hl = Wq.shape[1] // DH

    x2 = x[0].astype(bf)
    Q = jnp.dot(x2, Wq.astype(bf), preferred_element_type=jnp.float32)
    Q = Q.reshape(ROWS, hl, DH).astype(bf)
    K = lax.dynamic_slice_in_dim(K_ext[0], my * hl, hl, axis=1).astype(bf)
    V = lax.dynamic_slice_in_dim(V_ext[0], my * hl, hl, axis=1).astype(bf)

    def group(t):
        t = t.reshape(4, 4, 64, hl, DH)
        return t.transpose(1, 0, 2, 3, 4).reshape(4, 256, hl, DH)

    Qg, Kg, Vg = group(Q), group(K), group(V)
    scores = (
        jnp.einsum("gihd,gjhd->ghij", Qg, Kg, preferred_element_type=jnp.float32)
        * SCALE
    )
    w = jax.nn.softmax(scores, axis=-1).astype(bf)
    ctx = jnp.einsum("ghij,gjhd->gihd", w, Vg, preferred_element_type=jnp.float32)
    ctx = (
        ctx.reshape(4, 4, 64, hl, DH)
        .transpose(1, 0, 2, 3, 4)
        .reshape(ROWS, hl * DH)
        .astype(bf)
    )
    partial = jnp.dot(ctx, Wo.astype(bf), preferred_element_type=jnp.float32)

    out = _hier_allreduce(partial)
    return out.reshape(1, ROWS, ROWS)
